# Initial kernel scaffold; baseline (speedup 1.0000x reference)
import jax
import jax.numpy as jnp
from jax.experimental import pallas as pl


def kernel(in_tensor, conditioning_code, tables):
    raise NotImplementedError("write your pallas kernel here")


# speedup vs baseline: 1.5834x; 1.5834x over previous
import jax
import jax.numpy as jnp
from jax import lax
from jax.experimental import pallas as pl
from jax.experimental.pallas import tpu as pltpu
from jax.experimental.pallas import tpu_sc as plsc

N_TABLES = 4
N_LEVELS = 16
F_PER_LEVEL = 2
LOG2_HASH = 19
HASH_SIZE = 2 ** LOG2_HASH
MASK = HASH_SIZE - 1
BASE_RES = 16
PER_LEVEL_SCALE = 1.4472692012786865
N_POINTS = 131072
PRIME1 = 2654435761
PRIME2 = 805459861

NC = 2
NS = 16
NW = NC * NS
PPW = N_POINTS // NW
C = 16
NCHUNK = PPW // C
GC = 8
NGRP = NCHUNK // GC
ROWS = N_LEVELS * 8 * C
D_OUT = N_LEVELS * F_PER_LEVEL

import numpy as _np
RES = [int(_np.floor(BASE_RES * (PER_LEVEL_SCALE ** l))) for l in range(N_LEVELS)]

_P1 = PRIME1 - (1 << 32)
_P2 = PRIME2

_DNUMS = lax.GatherDimensionNumbers(
    offset_dims=(), collapsed_slice_dims=(0,), start_index_map=(0,))


def _vperm(v, idx16):
    return lax.gather(v, idx16[:, None], _DNUMS, (1,),
                      mode=lax.GatherScatterMode.PROMISE_IN_BOUNDS)


def _body(xs_h, ys_h, zs_h, c0_h, c1_h, c2_h, c3_h, tab_h, out_h,
          xs_v, ys_v, zs_v, c0_v, c1_v, c2_v, c3_v,
          idx_v, bw_v, rows_v, crep_v, out_v, sem):
    wid = lax.axis_index("s") * NC + lax.axis_index("c")
    base0 = wid * PPW
    pltpu.sync_copy(xs_h.at[pl.ds(base0, PPW)], xs_v)
    pltpu.sync_copy(ys_h.at[pl.ds(base0, PPW)], ys_v)
    pltpu.sync_copy(zs_h.at[pl.ds(base0, PPW)], zs_v)
    pltpu.sync_copy(c0_h.at[pl.ds(base0, PPW)], c0_v)
    pltpu.sync_copy(c1_h.at[pl.ds(base0, PPW)], c1_v)
    pltpu.sync_copy(c2_h.at[pl.ds(base0, PPW)], c2_v)
    pltpu.sync_copy(c3_h.at[pl.ds(base0, PPW)], c3_v)

    iota16 = lax.iota(jnp.int32, 16)
    splat = [jnp.full((16,), p, dtype=jnp.int32) for p in range(C)]
    sh2 = (iota16 + 2) & 15
    sh4 = (iota16 + 4) & 15
    posbase = (iota16 & 1) * 128
    m2 = iota16 < 2
    crep_pos = iota16 * 8

    def _do_chunk(g, i):
        cb = g * C
        col = i * C
        x = xs_v[pl.ds(cb, C)]
        y = ys_v[pl.ds(cb, C)]
        z = zs_v[pl.ds(cb, C)]

        for l in range(N_LEVELS):
            res = jnp.float32(RES[l])
            px = x * res
            py = y * res
            pz = z * res
            ix = px.astype(jnp.int32)
            iy = py.astype(jnp.int32)
            iz = pz.astype(jnp.int32)
            wx = px - ix.astype(jnp.float32)
            wy = py - iy.astype(jnp.float32)
            wz = pz - iz.astype(jnp.float32)
            ox = jnp.float32(1.0) - wx
            oy = jnp.float32(1.0) - wy
            oz = jnp.float32(1.0) - wz
            hy0 = iy * _P1
            hz0 = iz * _P2
            hy1 = (iy + 1) * _P1
            hz1 = (iz + 1) * _P2
            lbase = l * HASH_SIZE
            for c in range(8):
                bx, by, bz = c & 1, (c >> 1) & 1, (c >> 2) & 1
                cx = ix + 1 if bx else ix
                h = cx ^ (hy1 if by else hy0) ^ (hz1 if bz else hz0)
                gidx = (h & MASK) + lbase
                bw = (wx if bx else ox) * (wy if by else oy) * (wz if bz else oz)
                off = (l * 8 + c) * C
                idx_v[pl.ds(off, C)] = gidx
                bw_v[pl.ds(off, C)] = bw

        q = [c0_v[pl.ds(cb, C)], c1_v[pl.ds(cb, C)],
             c2_v[pl.ds(cb, C)], c3_v[pl.ds(cb, C)]]
        for t in range(4):
            for f in range(2):
                plsc.store_scatter(crep_v, [crep_pos + (2 * t + f)], q[t])

        descs = []
        for k in range(ROWS // 128):
            descs.append(pltpu.async_copy(
                tab_h.at[idx_v.at[pl.ds(k * 128, 128)]],
                rows_v.at[pl.ds(k * 128, 128), :], sem))
        for d in descs:
            d.wait()

        @pl.loop(0, N_LEVELS)
        def _lvl(l):
            acc = [jnp.zeros((16,), jnp.float32)] * C
            for c in range(8):
                rbase = (l * 8 + c) * C
                bw_lc = bw_v[pl.ds(rbase, C)]
                for p in range(C):
                    bwrep = _vperm(bw_lc, splat[p])
                    row = rows_v[rbase + p, :]
                    acc[p] = acc[p] + row * bwrep
            for p in range(C):
                cp = crep_v[pl.ds(p * 8, 16)]
                m = acc[p] * cp
                s1 = m + _vperm(m, sh2)
                s2 = s1 + _vperm(s1, sh4)
                pos = posbase + (2 * l * 128 + col + p)
                plsc.store_scatter(out_v, [pos], s2, mask=m2)

    @pl.loop(0, NGRP)
    def _grp(grp):
        @pl.loop(0, GC)
        def _chunk(i):
            _do_chunk(grp * GC + i, i)

        pltpu.sync_copy(out_v,
                        out_h.at[pl.ds((wid * NGRP + grp) * (D_OUT * 128),
                                       D_OUT * 128)])


@jax.jit
def _run(xs, ys, zs, c0, c1, c2, c3, tab):
    mesh = plsc.VectorSubcoreMesh(core_axis_name="c", subcore_axis_name="s")
    f = pl.kernel(
        _body,
        out_type=jax.ShapeDtypeStruct((N_POINTS * D_OUT,), jnp.float32),
        mesh=mesh,
        compiler_params=pltpu.CompilerParams(needs_layout_passes=False,
                                             use_tc_tiling_on_sc=False),
        scratch_types=[
            pltpu.VMEM((PPW,), jnp.float32),
            pltpu.VMEM((PPW,), jnp.float32),
            pltpu.VMEM((PPW,), jnp.float32),
            pltpu.VMEM((PPW,), jnp.float32),
            pltpu.VMEM((PPW,), jnp.float32),
            pltpu.VMEM((PPW,), jnp.float32),
            pltpu.VMEM((PPW,), jnp.float32),
            pltpu.VMEM((ROWS,), jnp.int32),
            pltpu.VMEM((ROWS,), jnp.float32),
            pltpu.VMEM((ROWS, 16), jnp.float32),
            pltpu.VMEM((256,), jnp.float32),
            pltpu.VMEM((D_OUT * 128,), jnp.float32),
            pltpu.SemaphoreType.DMA,
        ],
    )
    return f(xs, ys, zs, c0, c1, c2, c3, tab)


def kernel(in_tensor, conditioning_code, tables):
    xs = in_tensor[:, 0]
    ys = in_tensor[:, 1]
    zs = in_tensor[:, 2]
    c0 = conditioning_code[:, 0]
    c1 = conditioning_code[:, 1]
    c2 = conditioning_code[:, 2]
    c3 = conditioning_code[:, 3]
    t8 = jnp.transpose(tables, (1, 2, 0, 3)).reshape(N_LEVELS * HASH_SIZE,
                                                     N_TABLES * F_PER_LEVEL)
    t8n = jnp.concatenate([t8[1:], t8[:1]], axis=0)
    tab = jnp.concatenate([t8, t8n], axis=1)
    out = _run(xs, ys, zs, c0, c1, c2, c3, tab)
    out = out.reshape(NW, NGRP, D_OUT, 128).transpose(0, 1, 3, 2)
    return out.reshape(N_POINTS, D_OUT)
